# baseline (speedup 1.0000x reference)
import functools

import jax
import jax.numpy as jnp
from jax import lax
from jax.experimental import pallas as pl
from jax.experimental.pallas import tpu as pltpu
from jax.experimental.pallas import tpu_sc as plsc

N = 2048
D = 768
E = 8
F = 256
F2 = 2 * F
T = 256
NP = 4096 + E * T
G = NP // T
NW = 32
TPW = N // NW


def _router_body(x_ref, rs_ref, gwt_ref, pes_ref, posw_ref, wexp_ref, tetv_ref):
    xt = x_ref[...]
    var = jnp.mean(xt * xt, axis=1, keepdims=True)
    xn = xt * lax.rsqrt(var + 1e-6)
    h = xn * (D ** -0.5) * rs_ref[...]
    logits = jnp.dot(h, gwt_ref[...], preferred_element_type=jnp.float32)
    mx = jnp.max(logits, axis=1, keepdims=True)
    exl = jnp.exp(logits - mx)
    probs = exl / jnp.sum(exl, axis=1, keepdims=True)
    m1 = jnp.max(probs, axis=1, keepdims=True)
    m2 = jnp.max(jnp.where(probs >= m1, -jnp.inf, probs), axis=1, keepdims=True)
    sel = (probs >= m2).astype(jnp.float32)
    wfull = (probs * sel / (m1 + m2)) * pes_ref[...]

    BB = 256
    ri = lax.broadcasted_iota(jnp.int32, (BB, BB), 0)
    ci = lax.broadcasted_iota(jnp.int32, (BB, BB), 1)
    tril = (ci < ri).astype(jnp.float32)
    run = jnp.zeros((1, E), jnp.float32)
    blocks = []
    for b in range(N // BB):
        mb = sel[b * BB:(b + 1) * BB]
        blocks.append(jnp.dot(tril, mb, preferred_element_type=jnp.float32) + run)
        run = run + jnp.sum(mb, axis=0, keepdims=True)
    exc = jnp.concatenate(blocks, axis=0)

    pad = ((run.astype(jnp.int32) + (T - 1)) // T) * T
    jj = lax.broadcasted_iota(jnp.int32, (E, E), 0)
    ii = lax.broadcasted_iota(jnp.int32, (E, E), 1)
    supper = (jj < ii).astype(jnp.float32)
    pad_off = jnp.dot(pad.astype(jnp.float32), supper,
                      preferred_element_type=jnp.float32)
    pos = pad_off + exc

    lanes_before = jnp.dot(sel, supper, preferred_element_type=jnp.float32)
    fm = sel * (lanes_before == 0.0)
    sm = sel * (lanes_before == 1.0)
    posa = jnp.sum(fm * pos, axis=1, keepdims=True)
    posb = jnp.sum(sm * pos, axis=1, keepdims=True)
    wa = jnp.sum(fm * wfull, axis=1, keepdims=True)
    wb = jnp.sum(sm * wfull, axis=1, keepdims=True)
    posw_ref[...] = jnp.concatenate([posa, posb, wa, wb], axis=1)
    wexp_ref[...] = jnp.concatenate([
        jnp.broadcast_to(wa, (N, 16)), jnp.broadcast_to(wb, (N, 16))], axis=1)
    gt = lax.broadcasted_iota(jnp.int32, (32, 1), 0).astype(jnp.float32) * float(T)
    off_b = jnp.broadcast_to(pad_off, (32, E))
    te = jnp.sum((off_b <= gt).astype(jnp.int32), axis=1, keepdims=True) - 1
    total = jnp.sum(pad.astype(jnp.float32), axis=1, keepdims=True)
    tv = (gt < total).astype(jnp.int32)
    tetv_ref[...] = jnp.concatenate([te, tv], axis=1)


def _run_router(x2, router_scale, gate_w, per_expert_scale):
    return pl.pallas_call(
        _router_body,
        grid=(1,),
        in_specs=[
            pl.BlockSpec((N, D), lambda i: (0, 0)),
            pl.BlockSpec((1, D), lambda i: (0, 0)),
            pl.BlockSpec((D, E), lambda i: (0, 0)),
            pl.BlockSpec((1, E), lambda i: (0, 0)),
        ],
        out_specs=[
            pl.BlockSpec((N, 4), lambda i: (0, 0)),
            pl.BlockSpec((N, 32), lambda i: (0, 0)),
            pl.BlockSpec((32, 2), lambda i: (0, 0)),
        ],
        out_shape=[
            jax.ShapeDtypeStruct((N, 4), jnp.float32),
            jax.ShapeDtypeStruct((N, 32), jnp.float32),
            jax.ShapeDtypeStruct((32, 2), jnp.int32),
        ],
        compiler_params=pltpu.CompilerParams(
            dimension_semantics=("arbitrary",),
            vmem_limit_bytes=100 * 1024 * 1024,
        ),
    )(x2, router_scale.reshape(1, D), gate_w.T, per_expert_scale.reshape(1, E))


def _dispatch_sc(x2, posa, posb):
    mesh = plsc.VectorSubcoreMesh(core_axis_name="c", subcore_axis_name="s")

    @functools.partial(
        pl.kernel,
        mesh=mesh,
        out_type=jax.ShapeDtypeStruct((NP, D), jnp.float32),
        scratch_types=[
            pltpu.VMEM((TPW, D), jnp.float32),
            pltpu.VMEM((TPW,), jnp.int32),
            pltpu.VMEM((TPW,), jnp.int32),
            pltpu.SemaphoreType.DMA,
        ],
    )
    def disp(x_hbm, pa_hbm, pb_hbm, xs_hbm, rows_v, ia_v, ib_v, sem):
        wid = lax.axis_index("s") * 2 + lax.axis_index("c")
        base = wid * TPW
        pltpu.sync_copy(pa_hbm.at[pl.ds(base, TPW)], ia_v)
        pltpu.sync_copy(pb_hbm.at[pl.ds(base, TPW)], ib_v)
        pltpu.sync_copy(x_hbm.at[pl.ds(base, TPW)], rows_v)
        ca = pltpu.async_copy(rows_v, xs_hbm.at[ia_v], sem)
        ca.wait()
        cb = pltpu.async_copy(rows_v, xs_hbm.at[ib_v], sem)
        cb.wait()

    return disp(x2, posa, posb)


def _gmm_body(tetv_ref, xs_ref, gu_ref, dn_ref, ys_ref):
    g = pl.program_id(0)

    @pl.when(tetv_ref[g, 1] != 0)
    def _():
        xt = xs_ref[...]
        h2 = jnp.dot(xt, gu_ref[0], preferred_element_type=jnp.float32)
        gate = h2[:, :F]
        up = h2[:, F:]
        act = 0.5 * gate * (1.0 + lax.erf(gate * (2.0 ** -0.5))) * up
        ys_ref[...] = jnp.dot(act, dn_ref[0], preferred_element_type=jnp.float32)


def _run_gmm(tetv, xs, gate_up, down):
    grid_spec = pltpu.PrefetchScalarGridSpec(
        num_scalar_prefetch=1,
        grid=(G,),
        in_specs=[
            pl.BlockSpec((T, D), lambda g, tetv: (g, 0)),
            pl.BlockSpec((1, D, F2), lambda g, tetv: (tetv[g, 0], 0, 0)),
            pl.BlockSpec((1, F, D), lambda g, tetv: (tetv[g, 0], 0, 0)),
        ],
        out_specs=pl.BlockSpec((T, D), lambda g, tetv: (g, 0)),
    )
    return pl.pallas_call(
        _gmm_body,
        grid_spec=grid_spec,
        out_shape=jax.ShapeDtypeStruct((NP, D), jnp.float32),
        compiler_params=pltpu.CompilerParams(
            dimension_semantics=("arbitrary",),
            vmem_limit_bytes=100 * 1024 * 1024,
        ),
    )(tetv, xs, gate_up, down)


def _combine_sc(ys, posa, posb, wexp):
    mesh = plsc.VectorSubcoreMesh(core_axis_name="c", subcore_axis_name="s")

    @functools.partial(
        pl.kernel,
        mesh=mesh,
        out_type=jax.ShapeDtypeStruct((N, D), jnp.float32),
        scratch_types=[
            pltpu.VMEM((TPW, D), jnp.float32),
            pltpu.VMEM((TPW, D), jnp.float32),
            pltpu.VMEM((TPW,), jnp.int32),
            pltpu.VMEM((TPW,), jnp.int32),
            pltpu.VMEM((TPW, 32), jnp.float32),
            pltpu.SemaphoreType.DMA,
        ],
    )
    def comb(ys_hbm, pa_hbm, pb_hbm, wexp_hbm, out_hbm,
             bufa_v, bufb_v, ia_v, ib_v, wab_v, sem):
        wid = lax.axis_index("s") * 2 + lax.axis_index("c")
        base = wid * TPW
        pltpu.sync_copy(pa_hbm.at[pl.ds(base, TPW)], ia_v)
        pltpu.sync_copy(pb_hbm.at[pl.ds(base, TPW)], ib_v)
        pltpu.sync_copy(wexp_hbm.at[pl.ds(base, TPW)], wab_v)
        ga = pltpu.async_copy(ys_hbm.at[ia_v], bufa_v, sem)
        gb = pltpu.async_copy(ys_hbm.at[ib_v], bufb_v, sem)
        ga.wait()
        gb.wait()

        def row(r, carry):
            sa = wab_v[r, pl.ds(0, 16)]
            sb = wab_v[r, pl.ds(16, 16)]
            for c in range(D // 16):
                cs = pl.ds(c * 16, 16)
                bufa_v[r, cs] = sa * bufa_v[r, cs] + sb * bufb_v[r, cs]
            return carry

        lax.fori_loop(0, TPW, row, 0)
        pltpu.sync_copy(bufa_v, out_hbm.at[pl.ds(base, TPW)])

    return comb(ys, posa, posb, wexp)


def kernel(x, gate_up, down, per_expert_scale, router_scale, gate_w):
    x2 = x.reshape(N, D)
    posw, wexp, tetv = _run_router(x2, router_scale, gate_w, per_expert_scale)
    posa = posw[:, 0].astype(jnp.int32)
    posb = posw[:, 1].astype(jnp.int32)

    xs = _dispatch_sc(x2, posa, posb)
    ys = _run_gmm(tetv, xs, gate_up, down)
    out = _combine_sc(ys, posa, posb, wexp)
    return out.reshape(x.shape)


# speedup vs baseline: 1.4511x; 1.4511x over previous
import functools

import jax
import jax.numpy as jnp
from jax.experimental import pallas as pl
from jax.experimental.pallas import tpu as pltpu


def _moe_body(x_ref, gu_ref, dn_ref, pes_ref, rs_ref, gwt_ref, o_ref, *, E, F, D):
    xt = x_ref[...]
    var = jnp.mean(xt * xt, axis=1, keepdims=True)
    xn = xt * jax.lax.rsqrt(var + 1e-6)
    h = xn * (D ** -0.5) * rs_ref[...]
    logits = jnp.dot(h, gwt_ref[...], preferred_element_type=jnp.float32)
    mx = jnp.max(logits, axis=1, keepdims=True)
    ex = jnp.exp(logits - mx)
    probs = ex / jnp.sum(ex, axis=1, keepdims=True)
    m1 = jnp.max(probs, axis=1, keepdims=True)
    m2 = jnp.max(jnp.where(probs >= m1, -jnp.inf, probs), axis=1, keepdims=True)
    wsel = jnp.where(probs >= m2, probs, 0.0)
    wmask = (wsel / (m1 + m2)) * pes_ref[...]
    h2 = jnp.dot(xt, gu_ref[...], preferred_element_type=jnp.float32)
    acts = []
    for e in range(E):
        gate = h2[:, e * 2 * F:e * 2 * F + F]
        up = h2[:, e * 2 * F + F:(e + 1) * 2 * F]
        act = 0.5 * gate * (1.0 + jax.lax.erf(gate * (2.0 ** -0.5))) * up
        acts.append(act * wmask[:, e:e + 1])
    acts = jnp.concatenate(acts, axis=1)
    o_ref[...] = jnp.dot(acts, dn_ref[...], preferred_element_type=jnp.float32)


def kernel(x, gate_up, down, per_expert_scale, router_scale, gate_w):
    B, L, D = x.shape
    E, _, F2 = gate_up.shape
    F = F2 // 2
    N = B * L
    x2 = x.reshape(N, D)
    gu2 = gate_up.transpose(1, 0, 2).reshape(D, E * F2)
    dn2 = down.reshape(E * F, D)
    gate_wT = gate_w.T
    pes = per_expert_scale.reshape(1, E)
    rs = router_scale.reshape(1, D)

    TT = 512
    grid = (N // TT,)
    out = pl.pallas_call(
        functools.partial(_moe_body, E=E, F=F, D=D),
        grid=grid,
        in_specs=[
            pl.BlockSpec((TT, D), lambda i: (i, 0)),
            pl.BlockSpec((D, E * F2), lambda i: (0, 0)),
            pl.BlockSpec((E * F, D), lambda i: (0, 0)),
            pl.BlockSpec((1, E), lambda i: (0, 0)),
            pl.BlockSpec((1, D), lambda i: (0, 0)),
            pl.BlockSpec((D, E), lambda i: (0, 0)),
        ],
        out_specs=pl.BlockSpec((TT, D), lambda i: (i, 0)),
        out_shape=jax.ShapeDtypeStruct((N, D), jnp.float32),
        compiler_params=pltpu.CompilerParams(
            dimension_semantics=("arbitrary",),
            vmem_limit_bytes=100 * 1024 * 1024,
        ),
    )(x2, gu2, dn2, pes, rs, gate_wT)
    return out.reshape(B, L, D)
